# baseline (speedup 1.0000x reference)
import dataclasses
import functools

import jax
import jax.numpy as jnp
from jax import lax
from jax.experimental import pallas as pl
from jax.experimental.pallas import tpu as pltpu
from jax.experimental.pallas import tpu_sc as plsc

_K = 32
_LANES = 16
_TB = 400


def _sc_maxpool(inp, idx):
    c, n1 = inp.shape
    n2 = idx.shape[0]
    info = plsc.get_sparse_core_info()
    nc = info.num_cores
    nw = nc * info.num_subcores
    cpw = c // nw
    nb = n2 // _TB
    ub = _TB // _LANES

    mesh = plsc.VectorSubcoreMesh(core_axis_name="c", subcore_axis_name="s")
    cp = pltpu.CompilerParams()
    if "needs_layout_passes" in pltpu.CompilerParams.__dataclass_fields__:
        cp = dataclasses.replace(cp, needs_layout_passes=False)
    if "use_tc_tiling_on_sc" in pltpu.CompilerParams.__dataclass_fields__:
        cp = dataclasses.replace(cp, use_tc_tiling_on_sc=False)

    @functools.partial(
        pl.kernel,
        mesh=mesh,
        compiler_params=cp,
        out_type=jax.ShapeDtypeStruct((c, n2), jnp.float32),
        scratch_types=[
            pltpu.VMEM((cpw * n1,), jnp.float32),
            pltpu.VMEM((2, _TB, _K), jnp.int32),
            pltpu.VMEM((2, cpw, _TB), jnp.float32),
            pltpu.SemaphoreType.DMA,
            pltpu.SemaphoreType.DMA((2,)),
            pltpu.SemaphoreType.DMA((2,)),
        ],
    )
    def k(in_hbm, idx_hbm, out_hbm, tab_v, idx_v, obuf, tsem, isem, osem):
        w = lax.axis_index("s") * nc + lax.axis_index("c")
        c0 = w * cpw

        def idx_copy(blk, p):
            return pltpu.make_async_copy(
                idx_hbm.at[pl.ds(blk * _TB, _TB)], idx_v.at[p], isem.at[p])

        def out_copy(blk, p):
            return pltpu.make_async_copy(
                obuf.at[p], out_hbm.at[pl.ds(c0, cpw), pl.ds(blk * _TB, _TB)],
                osem.at[p])

        for cc in range(cpw):
            pltpu.make_async_copy(
                in_hbm.at[c0 + cc], tab_v.at[pl.ds(cc * n1, n1)], tsem).start()
        idx_copy(0, 0).start()
        idx_copy(1, 1).start()
        for cc in range(cpw):
            pltpu.make_async_copy(
                in_hbm.at[0], tab_v.at[pl.ds(0, n1)], tsem).wait()

        iota = lax.iota(jnp.int32, _LANES)
        kvecs = [jnp.full((_LANES,), kk, jnp.int32) for kk in range(_K)]

        def do_block(blk, p, drain, refill):
            idx_copy(blk, p).wait()
            if drain:
                out_copy(blk, p).wait()

            @pl.loop(0, ub)
            def _per_chunk(u):
                tvec = iota + u * _LANES
                iv = [plsc.load_gather(idx_v.at[p], [tvec, kvecs[kk]])
                      for kk in range(_K)]
                for cc in range(cpw):
                    off = cc * n1
                    acc = plsc.load_gather(tab_v, [iv[0] + off])
                    for kk in range(1, _K):
                        acc = jnp.maximum(
                            acc, plsc.load_gather(tab_v, [iv[kk] + off]))
                    obuf[p, cc, pl.ds(u * _LANES, _LANES)] = acc

            if refill:
                if isinstance(blk, int):
                    if blk + 2 < nb:
                        idx_copy(blk + 2, p).start()
                else:
                    @pl.when(blk + 2 < nb)
                    def _refill():
                        idx_copy(blk + 2, p).start()
            out_copy(blk, p).start()

        do_block(0, 0, False, True)
        do_block(1, 1, False, True)

        @pl.loop(2, nb - 1, step=2)
        def _per_pair(bi):
            do_block(bi, 0, True, True)
            do_block(bi + 1, 1, True, True)

        do_block(nb - 1, 0, True, False)

        out_copy(0, 0).wait()
        out_copy(1, 1).wait()

    return k(inp, idx)


def kernel(input, pts_x1, pts_x2, indices_xj_i_cache):
    u = input.reshape(-1, input.shape[-1])
    idx = indices_xj_i_cache.astype(jnp.int32)
    out = _sc_maxpool(u, idx)
    return (out.reshape(*input.shape[:-1], idx.shape[0]), pts_x2)


# speedup vs baseline: 1.8345x; 1.8345x over previous
import dataclasses
import functools

import jax
import jax.numpy as jnp
from jax import lax
from jax.experimental import pallas as pl
from jax.experimental.pallas import tpu as pltpu
from jax.experimental.pallas import tpu_sc as plsc

_K = 32
_LANES = 16
_TB = 400


def _sc_maxpool(inp, idx_t):
    c, n1 = inp.shape
    n2 = idx_t.shape[1]
    info = plsc.get_sparse_core_info()
    nc = info.num_cores
    nw = nc * info.num_subcores
    cpw = c // nw
    nb = n2 // _TB
    ub = _TB // _LANES

    mesh = plsc.VectorSubcoreMesh(core_axis_name="c", subcore_axis_name="s")
    cp = pltpu.CompilerParams()
    if "needs_layout_passes" in pltpu.CompilerParams.__dataclass_fields__:
        cp = dataclasses.replace(cp, needs_layout_passes=False)
    if "use_tc_tiling_on_sc" in pltpu.CompilerParams.__dataclass_fields__:
        cp = dataclasses.replace(cp, use_tc_tiling_on_sc=False)

    @functools.partial(
        pl.kernel,
        mesh=mesh,
        compiler_params=cp,
        out_type=jax.ShapeDtypeStruct((c, n2), jnp.float32),
        scratch_types=[
            pltpu.VMEM((cpw * n1,), jnp.float32),
            pltpu.VMEM((2, _K, _TB), jnp.int32),
            pltpu.VMEM((2, cpw, _TB), jnp.float32),
            pltpu.SemaphoreType.DMA,
            pltpu.SemaphoreType.DMA((2,)),
            pltpu.SemaphoreType.DMA((2,)),
        ],
    )
    def k(in_hbm, idx_hbm, out_hbm, tab_v, idx_v, obuf, tsem, isem, osem):
        w = lax.axis_index("s") * nc + lax.axis_index("c")
        c0 = w * cpw

        def idx_copy(blk, p):
            return pltpu.make_async_copy(
                idx_hbm.at[:, pl.ds(blk * _TB, _TB)], idx_v.at[p], isem.at[p])

        def out_copy(blk, p):
            return pltpu.make_async_copy(
                obuf.at[p], out_hbm.at[pl.ds(c0, cpw), pl.ds(blk * _TB, _TB)],
                osem.at[p])

        for cc in range(cpw):
            pltpu.make_async_copy(
                in_hbm.at[c0 + cc], tab_v.at[pl.ds(cc * n1, n1)], tsem).start()
        idx_copy(0, 0).start()
        idx_copy(1, 1).start()
        for cc in range(cpw):
            pltpu.make_async_copy(
                in_hbm.at[0], tab_v.at[pl.ds(0, n1)], tsem).wait()

        def do_block(blk, p, drain, refill):
            idx_copy(blk, p).wait()
            if drain:
                out_copy(blk, p).wait()

            @pl.loop(0, ub)
            def _per_chunk(u):
                sl = pl.ds(u * _LANES, _LANES)
                iv = [idx_v[p, kk, sl] for kk in range(_K)]
                for cc in range(cpw):
                    off = cc * n1
                    acc = plsc.load_gather(tab_v, [iv[0] + off])
                    for kk in range(1, _K):
                        acc = jnp.maximum(
                            acc, plsc.load_gather(tab_v, [iv[kk] + off]))
                    obuf[p, cc, pl.ds(u * _LANES, _LANES)] = acc

            if refill:
                if isinstance(blk, int):
                    if blk + 2 < nb:
                        idx_copy(blk + 2, p).start()
                else:
                    @pl.when(blk + 2 < nb)
                    def _refill():
                        idx_copy(blk + 2, p).start()
            out_copy(blk, p).start()

        do_block(0, 0, False, True)
        do_block(1, 1, False, True)

        @pl.loop(2, nb - 1, step=2)
        def _per_pair(bi):
            do_block(bi, 0, True, True)
            do_block(bi + 1, 1, True, True)

        do_block(nb - 1, 0, True, False)

        out_copy(0, 0).wait()
        out_copy(1, 1).wait()

    return k(inp, idx_t)


def kernel(input, pts_x1, pts_x2, indices_xj_i_cache):
    u = input.reshape(-1, input.shape[-1])
    idx_t = indices_xj_i_cache.astype(jnp.int32).T
    out = _sc_maxpool(u, idx_t)
    return (out.reshape(*input.shape[:-1], idx_t.shape[1]), pts_x2)


# speedup vs baseline: 2.2580x; 1.2308x over previous
import dataclasses
import functools

import jax
import jax.numpy as jnp
from jax import lax
from jax.experimental import pallas as pl
from jax.experimental.pallas import tpu as pltpu
from jax.experimental.pallas import tpu_sc as plsc

_K = 32
_KW = _K // 2
_LANES = 16
_TB = 400


def _sc_maxpool(tpk, ipk, c, n1, n2):
    info = plsc.get_sparse_core_info()
    nc = info.num_cores
    nw = nc * info.num_subcores
    cpw = c // nw
    wpw = cpw // 2
    nb = n2 // _TB
    ub = _TB // _LANES

    mesh = plsc.VectorSubcoreMesh(core_axis_name="c", subcore_axis_name="s")
    cp = pltpu.CompilerParams()
    if "needs_layout_passes" in pltpu.CompilerParams.__dataclass_fields__:
        cp = dataclasses.replace(cp, needs_layout_passes=False)
    if "use_tc_tiling_on_sc" in pltpu.CompilerParams.__dataclass_fields__:
        cp = dataclasses.replace(cp, use_tc_tiling_on_sc=False)

    @functools.partial(
        pl.kernel,
        mesh=mesh,
        compiler_params=cp,
        out_type=jax.ShapeDtypeStruct((c, n2), jnp.float32),
        scratch_types=[
            pltpu.VMEM((wpw * n1,), jnp.int32),
            pltpu.VMEM((2, _KW, _TB), jnp.int32),
            pltpu.VMEM((2, cpw, _TB), jnp.float32),
            pltpu.SemaphoreType.DMA,
            pltpu.SemaphoreType.DMA((2,)),
            pltpu.SemaphoreType.DMA((2,)),
        ],
    )
    def k(tab_hbm, idx_hbm, out_hbm, tab_v, idx_v, obuf, tsem, isem, osem):
        w = lax.axis_index("s") * nc + lax.axis_index("c")
        c0 = w * cpw
        r0 = w * wpw

        def idx_copy(blk, p):
            return pltpu.make_async_copy(
                idx_hbm.at[:, pl.ds(blk * _TB, _TB)], idx_v.at[p], isem.at[p])

        def out_copy(blk, p):
            return pltpu.make_async_copy(
                obuf.at[p], out_hbm.at[pl.ds(c0, cpw), pl.ds(blk * _TB, _TB)],
                osem.at[p])

        for rr in range(wpw):
            pltpu.make_async_copy(
                tab_hbm.at[r0 + rr], tab_v.at[pl.ds(rr * n1, n1)], tsem).start()
        idx_copy(0, 0).start()
        idx_copy(1, 1).start()
        for rr in range(wpw):
            pltpu.make_async_copy(
                tab_hbm.at[0], tab_v.at[pl.ds(0, n1)], tsem).wait()

        def do_block(blk, p, drain, refill):
            idx_copy(blk, p).wait()
            if drain:
                out_copy(blk, p).wait()

            @pl.loop(0, ub)
            def _per_chunk(u):
                sl = pl.ds(u * _LANES, _LANES)
                iv = []
                for jj in range(_KW):
                    packed = idx_v[p, jj, sl]
                    iv.append(jnp.bitwise_and(packed, 0xFFFF))
                    iv.append(lax.shift_right_logical(packed, 16))
                for rr in range(wpw):
                    off = rr * n1
                    subs = []
                    for s0 in range(0, _K, 8):
                        acc = plsc.bitcast(
                            plsc.load_gather(tab_v, [iv[s0] + off]),
                            jnp.bfloat16)
                        for kk in range(s0 + 1, s0 + 8):
                            g = plsc.bitcast(
                                plsc.load_gather(tab_v, [iv[kk] + off]),
                                jnp.bfloat16)
                            acc = jnp.maximum(acc, g)
                        subs.append(acc)
                    acc = jnp.maximum(jnp.maximum(subs[0], subs[1]),
                                      jnp.maximum(subs[2], subs[3]))
                    lo, hi = plsc.unpack(acc, format=plsc.PackFormat.INTERLEAVED)
                    obuf[p, 2 * rr, sl] = lo
                    obuf[p, 2 * rr + 1, sl] = hi

            if refill:
                if isinstance(blk, int):
                    if blk + 2 < nb:
                        idx_copy(blk + 2, p).start()
                else:
                    @pl.when(blk + 2 < nb)
                    def _refill():
                        idx_copy(blk + 2, p).start()
            out_copy(blk, p).start()

        do_block(0, 0, False, True)
        do_block(1, 1, False, True)

        @pl.loop(2, nb - 1, step=2)
        def _per_pair(bi):
            do_block(bi, 0, True, True)
            do_block(bi + 1, 1, True, True)

        do_block(nb - 1, 0, True, False)

        out_copy(0, 0).wait()
        out_copy(1, 1).wait()

    return k(tpk, ipk)


def kernel(input, pts_x1, pts_x2, indices_xj_i_cache):
    u = input.reshape(-1, input.shape[-1])
    c, n1 = u.shape
    n2, k = indices_xj_i_cache.shape
    tb = u.astype(jnp.bfloat16).reshape(c // 2, 2, n1).transpose(0, 2, 1)
    tpk = lax.bitcast_convert_type(tb, jnp.int32)
    ii = indices_xj_i_cache.astype(jnp.int16).reshape(n2, k // 2, 2)
    ipk = lax.bitcast_convert_type(ii, jnp.int32).T
    out = _sc_maxpool(tpk, ipk, c, n1, n2)
    return (out.reshape(*input.shape[:-1], n2), pts_x2)


# speedup vs baseline: 2.8344x; 1.2553x over previous
import dataclasses
import functools

import jax
import jax.numpy as jnp
from jax import lax
from jax.experimental import pallas as pl
from jax.experimental.pallas import tpu as pltpu
from jax.experimental.pallas import tpu_sc as plsc

_K = 32
_KW = _K // 2
_LANES = 16
_TB = 400


def _sc_maxpool(tpk, ipk, c, n1, n2):
    info = plsc.get_sparse_core_info()
    nc = info.num_cores
    nw = nc * info.num_subcores
    cpw = c // nw
    wpw = cpw // 2
    h = c // 2
    nb = n2 // _TB
    ub = _TB // _LANES

    mesh = plsc.VectorSubcoreMesh(core_axis_name="c", subcore_axis_name="s")
    cp = pltpu.CompilerParams()
    if "needs_layout_passes" in pltpu.CompilerParams.__dataclass_fields__:
        cp = dataclasses.replace(cp, needs_layout_passes=False)
    if "use_tc_tiling_on_sc" in pltpu.CompilerParams.__dataclass_fields__:
        cp = dataclasses.replace(cp, use_tc_tiling_on_sc=False)

    @functools.partial(
        pl.kernel,
        mesh=mesh,
        compiler_params=cp,
        out_type=jax.ShapeDtypeStruct((c, n2), jnp.float32),
        scratch_types=[
            pltpu.VMEM((wpw * n1,), jnp.int32),
            pltpu.VMEM((2, _KW, _TB), jnp.int32),
            pltpu.VMEM((2, cpw, _TB), jnp.float32),
            pltpu.SemaphoreType.DMA,
            pltpu.SemaphoreType.DMA((2,)),
            pltpu.SemaphoreType.DMA((2, 2)),
        ],
    )
    def k(tab_hbm, idx_hbm, out_hbm, tab_v, idx_v, obuf, tsem, isem, osem):
        w = lax.axis_index("s") * nc + lax.axis_index("c")
        r0 = w * wpw

        def idx_copy(blk, p):
            return pltpu.make_async_copy(
                idx_hbm.at[:, pl.ds(blk * _TB, _TB)], idx_v.at[p], isem.at[p])

        def out_copies(blk, p):
            tgt = pl.ds(blk * _TB, _TB)
            return [
                pltpu.make_async_copy(
                    obuf.at[p, pl.ds(0, wpw)],
                    out_hbm.at[pl.ds(r0, wpw), tgt], osem.at[p, 0]),
                pltpu.make_async_copy(
                    obuf.at[p, pl.ds(wpw, wpw)],
                    out_hbm.at[pl.ds(h + r0, wpw), tgt], osem.at[p, 1]),
            ]

        for rr in range(wpw):
            pltpu.make_async_copy(
                tab_hbm.at[r0 + rr], tab_v.at[pl.ds(rr * n1, n1)], tsem).start()
        idx_copy(0, 0).start()
        idx_copy(1, 1).start()
        for rr in range(wpw):
            pltpu.make_async_copy(
                tab_hbm.at[0], tab_v.at[pl.ds(0, n1)], tsem).wait()

        def do_block(blk, p, drain, refill):
            idx_copy(blk, p).wait()
            if drain:
                for cp_ in out_copies(blk, p):
                    cp_.wait()

            @pl.loop(0, ub)
            def _per_chunk(u):
                sl = pl.ds(u * _LANES, _LANES)
                iv = []
                for jj in range(_KW):
                    packed = idx_v[p, jj, sl]
                    iv.append(jnp.bitwise_and(packed, 0xFFFF))
                    iv.append(lax.shift_right_logical(packed, 16))
                for rr in range(wpw):
                    off = rr * n1
                    subs = []
                    for s0 in range(0, _K, 8):
                        acc = plsc.bitcast(
                            plsc.load_gather(tab_v, [iv[s0] + off]),
                            jnp.bfloat16)
                        for kk in range(s0 + 1, s0 + 8):
                            g = plsc.bitcast(
                                plsc.load_gather(tab_v, [iv[kk] + off]),
                                jnp.bfloat16)
                            acc = jnp.maximum(acc, g)
                        subs.append(acc)
                    acc = jnp.maximum(jnp.maximum(subs[0], subs[1]),
                                      jnp.maximum(subs[2], subs[3]))
                    lo, hi = plsc.unpack(acc, format=plsc.PackFormat.INTERLEAVED)
                    obuf[p, rr, sl] = lo
                    obuf[p, wpw + rr, sl] = hi

            if refill:
                if isinstance(blk, int):
                    if blk + 2 < nb:
                        idx_copy(blk + 2, p).start()
                else:
                    @pl.when(blk + 2 < nb)
                    def _refill():
                        idx_copy(blk + 2, p).start()
            for cp_ in out_copies(blk, p):
                cp_.start()

        do_block(0, 0, False, True)
        do_block(1, 1, False, True)

        @pl.loop(2, nb - 1, step=2)
        def _per_pair(bi):
            do_block(bi, 0, True, True)
            do_block(bi + 1, 1, True, True)

        do_block(nb - 1, 0, True, False)

        for cp_ in out_copies(0, 0):
            cp_.wait()
        for cp_ in out_copies(1, 1):
            cp_.wait()

    return k(tpk, ipk)


def kernel(input, pts_x1, pts_x2, indices_xj_i_cache):
    u = input.reshape(-1, input.shape[-1])
    c, n1 = u.shape
    n2, k = indices_xj_i_cache.shape
    h = c // 2
    lo = lax.bitcast_convert_type(
        u[:h].astype(jnp.bfloat16), jnp.uint16).astype(jnp.uint32)
    hi = lax.bitcast_convert_type(
        u[h:].astype(jnp.bfloat16), jnp.uint16).astype(jnp.uint32)
    tpk = lax.bitcast_convert_type(lo | (hi << 16), jnp.int32)
    ii = indices_xj_i_cache
    ipk = (ii[:, : k // 2] | (ii[:, k // 2:] << 16)).T
    out = _sc_maxpool(tpk, ipk, c, n1, n2)
    return (out.reshape(*input.shape[:-1], n2), pts_x2)


# speedup vs baseline: 2.8435x; 1.0032x over previous
import dataclasses
import functools

import jax
import jax.numpy as jnp
from jax import lax
from jax.experimental import pallas as pl
from jax.experimental.pallas import tpu as pltpu
from jax.experimental.pallas import tpu_sc as plsc

_K = 32
_KW = _K // 2
_LANES = 16
_TB = 400


def _sc_maxpool(tpk, ipk, c, n1, n2):
    info = plsc.get_sparse_core_info()
    nc = info.num_cores
    nw = nc * info.num_subcores
    cpw = c // nw
    wpw = cpw // 2
    h = c // 2
    nb = n2 // _TB
    ub = _TB // _LANES

    mesh = plsc.VectorSubcoreMesh(core_axis_name="c", subcore_axis_name="s")
    cp = pltpu.CompilerParams()
    if "needs_layout_passes" in pltpu.CompilerParams.__dataclass_fields__:
        cp = dataclasses.replace(cp, needs_layout_passes=False)
    if "use_tc_tiling_on_sc" in pltpu.CompilerParams.__dataclass_fields__:
        cp = dataclasses.replace(cp, use_tc_tiling_on_sc=False)

    @functools.partial(
        pl.kernel,
        mesh=mesh,
        compiler_params=cp,
        out_type=jax.ShapeDtypeStruct((c, n2), jnp.float32),
        scratch_types=[
            pltpu.VMEM((wpw * n1,), jnp.int32),
            pltpu.VMEM((2, _KW, _TB), jnp.int32),
            pltpu.VMEM((2, cpw, _TB), jnp.float32),
            pltpu.SemaphoreType.DMA,
            pltpu.SemaphoreType.DMA((2,)),
            pltpu.SemaphoreType.DMA((2, 2)),
        ],
    )
    def k(tab_hbm, idx_hbm, out_hbm, tab_v, idx_v, obuf, tsem, isem, osem):
        w = lax.axis_index("s") * nc + lax.axis_index("c")
        r0 = w * wpw

        def idx_copy(blk, p):
            return pltpu.make_async_copy(
                idx_hbm.at[:, pl.ds(blk * _TB, _TB)], idx_v.at[p], isem.at[p])

        def out_copies(blk, p):
            tgt = pl.ds(blk * _TB, _TB)
            return [
                pltpu.make_async_copy(
                    obuf.at[p, pl.ds(0, wpw)],
                    out_hbm.at[pl.ds(r0, wpw), tgt], osem.at[p, 0]),
                pltpu.make_async_copy(
                    obuf.at[p, pl.ds(wpw, wpw)],
                    out_hbm.at[pl.ds(h + r0, wpw), tgt], osem.at[p, 1]),
            ]

        for rr in range(wpw):
            pltpu.make_async_copy(
                tab_hbm.at[r0 + rr], tab_v.at[pl.ds(rr * n1, n1)], tsem).start()
        idx_copy(0, 0).start()
        idx_copy(1, 1).start()
        for rr in range(wpw):
            pltpu.make_async_copy(
                tab_hbm.at[0], tab_v.at[pl.ds(0, n1)], tsem).wait()

        def do_block(blk, p, drain, refill):
            idx_copy(blk, p).wait()
            if drain:
                for cp_ in out_copies(blk, p):
                    cp_.wait()

            @pl.loop(0, ub)
            def _per_chunk(u):
                sl = pl.ds(u * _LANES, _LANES)
                iv = []
                for jj in range(_KW):
                    packed = idx_v[p, jj, sl]
                    iv.append(jnp.bitwise_and(packed, 0xFFFF))
                    iv.append(lax.shift_right_logical(packed, 16))
                for rr in range(wpw):
                    tv = tab_v.at[pl.ds(rr * n1, n1)]
                    subs = []
                    for s0 in range(0, _K, 8):
                        acc = plsc.bitcast(
                            plsc.load_gather(tv, [iv[s0]]), jnp.bfloat16)
                        for kk in range(s0 + 1, s0 + 8):
                            g = plsc.bitcast(
                                plsc.load_gather(tv, [iv[kk]]), jnp.bfloat16)
                            acc = jnp.maximum(acc, g)
                        subs.append(acc)
                    acc = jnp.maximum(jnp.maximum(subs[0], subs[1]),
                                      jnp.maximum(subs[2], subs[3]))
                    lo, hi = plsc.unpack(acc, format=plsc.PackFormat.INTERLEAVED)
                    obuf[p, rr, sl] = lo
                    obuf[p, wpw + rr, sl] = hi

            if refill:
                if isinstance(blk, int):
                    if blk + 2 < nb:
                        idx_copy(blk + 2, p).start()
                else:
                    @pl.when(blk + 2 < nb)
                    def _refill():
                        idx_copy(blk + 2, p).start()
            for cp_ in out_copies(blk, p):
                cp_.start()

        do_block(0, 0, False, True)
        do_block(1, 1, False, True)

        @pl.loop(2, nb - 1, step=2)
        def _per_pair(bi):
            do_block(bi, 0, True, True)
            do_block(bi + 1, 1, True, True)

        do_block(nb - 1, 0, True, False)

        for cp_ in out_copies(0, 0):
            cp_.wait()
        for cp_ in out_copies(1, 1):
            cp_.wait()

    return k(tpk, ipk)


def kernel(input, pts_x1, pts_x2, indices_xj_i_cache):
    u = input.reshape(-1, input.shape[-1])
    c, n1 = u.shape
    n2, k = indices_xj_i_cache.shape
    h = c // 2
    lo = lax.bitcast_convert_type(
        u[:h].astype(jnp.bfloat16), jnp.uint16).astype(jnp.uint32)
    hi = lax.bitcast_convert_type(
        u[h:].astype(jnp.bfloat16), jnp.uint16).astype(jnp.uint32)
    tpk = lax.bitcast_convert_type(lo | (hi << 16), jnp.int32)
    ii = indices_xj_i_cache
    ipk = (ii[:, : k // 2] | (ii[:, k // 2:] << 16)).T
    out = _sc_maxpool(tpk, ipk, c, n1, n2)
    return (out.reshape(*input.shape[:-1], n2), pts_x2)


# speedup vs baseline: 2.9227x; 1.0279x over previous
import dataclasses
import functools

import jax
import jax.numpy as jnp
from jax import lax
from jax.experimental import pallas as pl
from jax.experimental.pallas import tpu as pltpu
from jax.experimental.pallas import tpu_sc as plsc

_K = 32
_KW = _K // 2
_LANES = 16
_TB = 400


def _sc_maxpool(tpk, ipk, c, n1, n2):
    info = plsc.get_sparse_core_info()
    nc = info.num_cores
    nw = nc * info.num_subcores
    cpw = c // nw
    wpw = cpw // 2
    h = c // 2
    nb = n2 // _TB
    ub = _TB // _LANES

    mesh = plsc.VectorSubcoreMesh(core_axis_name="c", subcore_axis_name="s")
    cp = pltpu.CompilerParams()
    if "needs_layout_passes" in pltpu.CompilerParams.__dataclass_fields__:
        cp = dataclasses.replace(cp, needs_layout_passes=False)
    if "use_tc_tiling_on_sc" in pltpu.CompilerParams.__dataclass_fields__:
        cp = dataclasses.replace(cp, use_tc_tiling_on_sc=False)

    @functools.partial(
        pl.kernel,
        mesh=mesh,
        compiler_params=cp,
        out_type=jax.ShapeDtypeStruct((c // 2, n2), jnp.int32),
        scratch_types=[
            pltpu.VMEM((wpw * n1,), jnp.int32),
            pltpu.VMEM((2, _KW, _TB), jnp.int32),
            pltpu.VMEM((2, wpw, _TB), jnp.int32),
            pltpu.SemaphoreType.DMA,
            pltpu.SemaphoreType.DMA((2,)),
            pltpu.SemaphoreType.DMA((2,)),
        ],
    )
    def k(tab_hbm, idx_hbm, out_hbm, tab_v, idx_v, obuf, tsem, isem, osem):
        w = lax.axis_index("s") * nc + lax.axis_index("c")
        r0 = w * wpw

        def idx_copy(blk, p):
            return pltpu.make_async_copy(
                idx_hbm.at[:, pl.ds(blk * _TB, _TB)], idx_v.at[p], isem.at[p])

        def out_copies(blk, p):
            return [pltpu.make_async_copy(
                obuf.at[p],
                out_hbm.at[pl.ds(r0, wpw), pl.ds(blk * _TB, _TB)],
                osem.at[p])]

        for rr in range(wpw):
            pltpu.make_async_copy(
                tab_hbm.at[r0 + rr], tab_v.at[pl.ds(rr * n1, n1)], tsem).start()
        idx_copy(0, 0).start()
        idx_copy(1, 1).start()
        for rr in range(wpw):
            pltpu.make_async_copy(
                tab_hbm.at[0], tab_v.at[pl.ds(0, n1)], tsem).wait()

        def do_block(blk, p, drain, refill):
            idx_copy(blk, p).wait()
            if drain:
                for cp_ in out_copies(blk, p):
                    cp_.wait()

            @pl.loop(0, ub)
            def _per_chunk(u):
                sl = pl.ds(u * _LANES, _LANES)
                iv = []
                for jj in range(_KW):
                    packed = idx_v[p, jj, sl]
                    iv.append(jnp.bitwise_and(packed, 0xFFFF))
                    iv.append(lax.shift_right_logical(packed, 16))
                for rr in range(wpw):
                    tv = tab_v.at[pl.ds(rr * n1, n1)]
                    subs = []
                    for s0 in range(0, _K, 8):
                        acc = plsc.bitcast(
                            plsc.load_gather(tv, [iv[s0]]), jnp.bfloat16)
                        for kk in range(s0 + 1, s0 + 8):
                            g = plsc.bitcast(
                                plsc.load_gather(tv, [iv[kk]]), jnp.bfloat16)
                            acc = jnp.maximum(acc, g)
                        subs.append(acc)
                    acc = jnp.maximum(jnp.maximum(subs[0], subs[1]),
                                      jnp.maximum(subs[2], subs[3]))
                    obuf[p, rr, sl] = plsc.bitcast(acc, jnp.int32)

            if refill:
                if isinstance(blk, int):
                    if blk + 2 < nb:
                        idx_copy(blk + 2, p).start()
                else:
                    @pl.when(blk + 2 < nb)
                    def _refill():
                        idx_copy(blk + 2, p).start()
            for cp_ in out_copies(blk, p):
                cp_.start()

        do_block(0, 0, False, True)
        do_block(1, 1, False, True)

        @pl.loop(2, nb - 1, step=2)
        def _per_pair(bi):
            do_block(bi, 0, True, True)
            do_block(bi + 1, 1, True, True)

        do_block(nb - 1, 0, True, False)

        for cp_ in out_copies(0, 0):
            cp_.wait()
        for cp_ in out_copies(1, 1):
            cp_.wait()

    return k(tpk, ipk)


def kernel(input, pts_x1, pts_x2, indices_xj_i_cache):
    u = input.reshape(-1, input.shape[-1])
    c, n1 = u.shape
    n2, k = indices_xj_i_cache.shape
    h = c // 2
    lo = lax.bitcast_convert_type(
        u[:h].astype(jnp.bfloat16), jnp.uint16).astype(jnp.uint32)
    hi = lax.bitcast_convert_type(
        u[h:].astype(jnp.bfloat16), jnp.uint16).astype(jnp.uint32)
    tpk = lax.bitcast_convert_type(lo | (hi << 16), jnp.int32)
    ii = indices_xj_i_cache
    ipk = (ii[:, : k // 2] | (ii[:, k // 2:] << 16)).T
    pk = _sc_maxpool(tpk, ipk, c, n1, n2)
    lo_f = lax.bitcast_convert_type(pk << 16, jnp.float32)
    hi_f = lax.bitcast_convert_type(
        jnp.bitwise_and(pk, jnp.int32(-65536)), jnp.float32)
    out = jnp.concatenate([lo_f, hi_f], axis=0)
    return (out.reshape(*input.shape[:-1], n2), pts_x2)
